# baseline (speedup 1.0000x reference)
import functools

import jax
import jax.numpy as jnp
from jax import lax
from jax.experimental import pallas as pl
from jax.experimental.pallas import tpu as pltpu
from jax.experimental.pallas import tpu_sc as plsc

N_OBJECTS = 1000000
EMBEDDING_DIM = 64
BATCH = 16384

_info = plsc.get_sparse_core_info()
_NC, _NS = _info.num_cores, _info.num_subcores
_NW = _NC * _NS
_B_PER_W = BATCH // _NW


def _embedding_kernel(idx_hbm, table_hbm, out_hbm, idx_v, rows_v, sem):
    wid = lax.axis_index("s") * _NC + lax.axis_index("c")
    base = wid * _B_PER_W
    pltpu.sync_copy(idx_hbm.at[pl.ds(base, _B_PER_W)], idx_v)
    pltpu.async_copy(table_hbm.at[idx_v], rows_v, sem).wait()
    pltpu.sync_copy(rows_v, out_hbm.at[pl.ds(base, _B_PER_W)])


@jax.jit
def kernel(o_idxs, table):
    mesh = plsc.VectorSubcoreMesh(core_axis_name="c", subcore_axis_name="s")
    run = functools.partial(
        pl.kernel,
        mesh=mesh,
        out_type=jax.ShapeDtypeStruct((BATCH, EMBEDDING_DIM), jnp.float32),
        scratch_types=[
            pltpu.VMEM((_B_PER_W,), jnp.int32),
            pltpu.VMEM((_B_PER_W, EMBEDDING_DIM), jnp.float32),
            pltpu.SemaphoreType.DMA,
        ],
        compiler_params=pltpu.CompilerParams(use_tc_tiling_on_sc=False),
    )(_embedding_kernel)
    return run(o_idxs.astype(jnp.int32), table)


# speedup vs baseline: 2.5510x; 2.5510x over previous
import functools

import jax
import jax.numpy as jnp
from jax import lax
from jax.experimental import pallas as pl
from jax.experimental.pallas import tpu as pltpu
from jax.experimental.pallas import tpu_sc as plsc

N_OBJECTS = 1000000
EMBEDDING_DIM = 64
BATCH = 16384
SUBLANES = 8

_info = plsc.get_sparse_core_info()
_NC, _NS = _info.num_cores, _info.num_subcores
_NW = _NC * _NS
_B_PER_W = BATCH // _NW
_GROUP = 16
_N_GROUPS = _B_PER_W // _GROUP
_DRAIN_LAG = 8


def _embedding_kernel(idx_hbm, table_hbm, out_hbm, idx_v, rows_v, sem):
    wid = lax.axis_index("s") * _NC + lax.axis_index("c")
    base = wid * _B_PER_W

    pltpu.sync_copy(idx_hbm.at[pl.ds(base, _B_PER_W)], idx_v)

    def group_body(g, _):
        v = idx_v[pl.ds(g * _GROUP, _GROUP)]
        t = lax.shift_right_logical(v, 3)
        s = v & 7
        for k in range(_GROUP):
            pltpu.async_copy(
                table_hbm.at[t[k], s[k]],
                rows_v.at[g * _GROUP + k],
                sem)

        @pl.when(g >= _DRAIN_LAG)
        def _():
            pltpu.make_async_copy(
                out_hbm.at[pl.ds(base, _GROUP)],
                rows_v.at[pl.ds(0, _GROUP)],
                sem).wait()

        return 0

    lax.fori_loop(0, _N_GROUPS, group_body, 0)

    pltpu.make_async_copy(
        out_hbm.at[pl.ds(base, _DRAIN_LAG * _GROUP)],
        rows_v.at[pl.ds(0, _DRAIN_LAG * _GROUP)],
        sem).wait()

    pltpu.sync_copy(rows_v, out_hbm.at[pl.ds(base, _B_PER_W)])


@jax.jit
def kernel(o_idxs, table):
    n_tiles = N_OBJECTS // SUBLANES
    table3 = table.reshape(n_tiles, SUBLANES, EMBEDDING_DIM)
    mesh = plsc.VectorSubcoreMesh(core_axis_name="c", subcore_axis_name="s")
    run = functools.partial(
        pl.kernel,
        mesh=mesh,
        out_type=jax.ShapeDtypeStruct((BATCH, EMBEDDING_DIM), jnp.float32),
        scratch_types=[
            pltpu.VMEM((_B_PER_W,), jnp.int32),
            pltpu.VMEM((_B_PER_W, EMBEDDING_DIM), jnp.float32),
            pltpu.SemaphoreType.DMA,
        ],
    )(_embedding_kernel)
    return run(o_idxs.astype(jnp.int32), table3)
